# Initial kernel scaffold; baseline (speedup 1.0000x reference)
import jax
import jax.numpy as jnp
from jax.experimental import pallas as pl


def kernel(x, edge_index, W1, b1, g1, be1, W2, b2, g2, be2, go, bo):
    raise NotImplementedError("write your pallas kernel here")


# speedup vs baseline: 9.8324x; 9.8324x over previous
import functools

import jax
import jax.numpy as jnp
from jax import lax
from jax.experimental import pallas as pl
from jax.experimental.pallas import tpu as pltpu
from jax.experimental.pallas import tpu_sc as plsc

N = 10000
E = 320000
D = 128
L = 3
BN_EPS = 1e-5

NC = 2
NS = 16
NW = NC * NS
EPT = E // NW
K = 40
CH = EPT // K
NBUF = 5
NP = 10240
RPT = NP // NS


def _sc_agg_body(h_hbm, z0_hbm, idx_hbm, out_hbm,
                 ibuf, rows_v, agg_sh, isem, gsem):
    cid = lax.axis_index("c")
    sid = lax.axis_index("s")
    wid = cid * NS + sid
    rs = sid * RPT

    @pl.when(cid == 0)
    def _():
        pltpu.sync_copy(h_hbm.at[pl.ds(rs, RPT)], agg_sh.at[pl.ds(rs, RPT)])

    @pl.when(cid != 0)
    def _():
        pltpu.sync_copy(z0_hbm, agg_sh.at[pl.ds(rs, RPT)])

    plsc.subcore_barrier()

    def _idx_start(c, b):
        pltpu.async_copy(idx_hbm.at[wid, c], ibuf.at[b], isem.at[b])

    def _idx_wait(b):
        pltpu.make_async_copy(idx_hbm.at[0, 0], ibuf.at[b], isem.at[b]).wait()

    def _gather_start(b):
        pltpu.async_copy(h_hbm.at[ibuf.at[b, 0]], rows_v.at[b], gsem.at[b])

    def _gather_wait(b):
        pltpu.make_async_copy(h_hbm.at[pl.ds(0, K)], rows_v.at[b],
                              gsem.at[b]).wait()

    def _scatter(b):
        pltpu.sync_copy(rows_v.at[b], agg_sh.at[ibuf.at[b, 1]], add=True)

    for j in range(NBUF):
        _idx_start(j, j)
    for j in range(NBUF - 1):
        _idx_wait(j)
        _gather_start(j)

    @pl.loop(0, CH - NBUF, step=NBUF)
    def _(cc):
        for b in range(NBUF):
            c = cc + b
            _gather_wait(b)
            _scatter(b)
            _idx_start(c + NBUF, b)
            bn = (b + NBUF - 1) % NBUF
            _idx_wait(bn)
            _gather_start(bn)

    for j in range(NBUF):
        b = (CH - NBUF + j) % NBUF
        _gather_wait(b)
        _scatter(b)
        if j == 0:
            bn = (b + NBUF - 1) % NBUF
            _idx_wait(bn)
            _gather_start(bn)

    plsc.subcore_barrier()
    pltpu.sync_copy(agg_sh.at[pl.ds(rs, RPT)],
                    out_hbm.at[cid, pl.ds(rs, RPT)])


@functools.lru_cache(maxsize=None)
def _sc_aggregate():
  return pl.kernel(
    _sc_agg_body,
    out_type=jax.ShapeDtypeStruct((NC, NP, D), jnp.float32),
    mesh=plsc.VectorSubcoreMesh(core_axis_name="c", subcore_axis_name="s",
                                num_cores=NC, num_subcores=NS),
    scratch_types=[
        pltpu.VMEM((NBUF, 2, K), jnp.int32),
        pltpu.VMEM((NBUF, K, D), jnp.float32),
        pltpu.VMEM_SHARED((NP, D), jnp.float32),
        pltpu.SemaphoreType.DMA((NBUF,)),
        pltpu.SemaphoreType.DMA((NBUF,)),
    ],
  )


def _bn(h, g, b):
    mean = jnp.mean(h, axis=0)
    var = jnp.mean((h - mean) ** 2, axis=0)
    return g * (h - mean) / jnp.sqrt(var + BN_EPS) + b


def _mlp_body(last, p_ref, w1_ref, b1_ref, g1_ref, be1_ref,
              w2_ref, b2_ref, g2_ref, be2_ref, go_ref, bo_ref, out_ref):
    z = p_ref[0, :N, :] + p_ref[1, :N, :]
    y = lax.dot_general(z, w1_ref[...], (((1,), (1,)), ((), ())),
                        preferred_element_type=jnp.float32) + b1_ref[...]
    y = jax.nn.relu(_bn(y, g1_ref[...], be1_ref[...]))
    t = lax.dot_general(y, w2_ref[...], (((1,), (1,)), ((), ())),
                        preferred_element_type=jnp.float32) + b2_ref[...]
    t = _bn(t, g2_ref[...], be2_ref[...])
    h = _bn(t, go_ref[...], bo_ref[...])
    if not last:
        h = jax.nn.relu(h)
    out_ref[...] = h


def _tc_mlp(p, w1, b1, g1, be1, w2, b2, g2, be2, go, bo, last):
    return pl.pallas_call(
        functools.partial(_mlp_body, last),
        out_shape=jax.ShapeDtypeStruct((N, D), jnp.float32),
    )(p, w1, b1, g1, be1, w2, b2, g2, be2, go, bo)


def kernel(x, edge_index, W1, b1, g1, be1, W2, b2, g2, be2, go, bo):
    idx = edge_index.astype(jnp.int32).reshape(2, NW, CH, K).transpose(1, 2, 0, 3)
    z0 = jnp.zeros((RPT, D), jnp.float32)
    h = x
    for i in range(L):
        hp = jnp.pad(h, ((0, NP - N), (0, 0)))
        p = _sc_aggregate()(hp, z0, idx)
        h = _tc_mlp(p, W1[i], b1[i], g1[i], be1[i], W2[i], b2[i],
                    g2[i], be2[i], go[i], bo[i], last=(i == L - 1))
    return h


# speedup vs baseline: 10.7196x; 1.0902x over previous
import functools

import jax
import jax.numpy as jnp
from jax import lax
from jax.experimental import pallas as pl
from jax.experimental.pallas import tpu as pltpu
from jax.experimental.pallas import tpu_sc as plsc

N = 10000
E = 320000
D = 128
L = 3
BN_EPS = 1e-5

NC = 2
NS = 16
NW = NC * NS
EPT = E // NW
K = 40
CH = EPT // K
NBUF = 5
NP = 10240
RPT = NP // NS


RB = 5
IBN = 10


def _sc_agg_body(h_hbm, z0_hbm, idx_hbm, out_hbm,
                 ibuf, rows_v, agg_sh, isem, gsem, ssem):
    cid = lax.axis_index("c")
    sid = lax.axis_index("s")
    wid = cid * NS + sid
    rs = sid * RPT

    @pl.when(cid == 0)
    def _():
        pltpu.sync_copy(h_hbm.at[pl.ds(rs, RPT)], agg_sh.at[pl.ds(rs, RPT)])

    @pl.when(cid != 0)
    def _():
        pltpu.sync_copy(z0_hbm, agg_sh.at[pl.ds(rs, RPT)])

    plsc.subcore_barrier()

    def _idx_start(c, s10):
        pltpu.async_copy(idx_hbm.at[wid, c], ibuf.at[s10], isem.at[s10])

    def _idx_wait(s10):
        pltpu.make_async_copy(idx_hbm.at[0, 0], ibuf.at[s10],
                              isem.at[s10]).wait()

    def _gather_start(s5, s10):
        pltpu.async_copy(h_hbm.at[ibuf.at[s10, 0]], rows_v.at[s5],
                         gsem.at[s5])

    def _gather_wait(s5):
        pltpu.make_async_copy(h_hbm.at[pl.ds(0, K)], rows_v.at[s5],
                              gsem.at[s5]).wait()

    def _scatter_start(s5, s10):
        pltpu.async_copy(rows_v.at[s5], agg_sh.at[ibuf.at[s10, 1]],
                         ssem.at[s5], add=True)

    def _scatter_wait(s5):
        pltpu.make_async_copy(rows_v.at[s5], agg_sh.at[pl.ds(0, K)],
                              ssem.at[s5]).wait()

    for j in range(8):
        _idx_start(j, j)
    for j in range(3):
        _idx_wait(j)
        _gather_start(j, j)
    for c in range(2):
        _gather_wait(c % RB)
        _scatter_start(c % RB, c % IBN)
        _idx_start(c + 8, (c + 8) % IBN)
        _idx_wait((c + 3) % IBN)
        _gather_start((c + 3) % RB, (c + 3) % IBN)

    @pl.loop(2, CH - 8, step=IBN)
    def _(cc):
        for b in range(IBN):
            c = cc + b
            s5, s10 = (2 + b) % RB, (2 + b) % IBN
            _gather_wait(s5)
            _scatter_start(s5, s10)
            _scatter_wait((s5 + 3) % RB)
            _idx_start(c + 8, (s10 + 8) % IBN)
            _idx_wait((s10 + 3) % IBN)
            _gather_start((s5 + 3) % RB, (s10 + 3) % IBN)

    for c in range(CH - 8, CH):
        _gather_wait(c % RB)
        _scatter_start(c % RB, c % IBN)
        _scatter_wait((c + 3) % RB)
        if c + 3 < CH:
            _idx_wait((c + 3) % IBN)
            _gather_start((c + 3) % RB, (c + 3) % IBN)
    for c in range(CH - 2, CH):
        _scatter_wait(c % RB)

    plsc.subcore_barrier()
    pltpu.sync_copy(agg_sh.at[pl.ds(rs, RPT)],
                    out_hbm.at[cid, pl.ds(rs, RPT)])


@functools.lru_cache(maxsize=None)
def _sc_aggregate():
  return pl.kernel(
    _sc_agg_body,
    out_type=jax.ShapeDtypeStruct((NC, NP, D), jnp.float32),
    mesh=plsc.VectorSubcoreMesh(core_axis_name="c", subcore_axis_name="s",
                                num_cores=NC, num_subcores=NS),
    scratch_types=[
        pltpu.VMEM((IBN, 2, K), jnp.int32),
        pltpu.VMEM((RB, K, D), jnp.float32),
        pltpu.VMEM_SHARED((NP, D), jnp.float32),
        pltpu.SemaphoreType.DMA((IBN,)),
        pltpu.SemaphoreType.DMA((RB,)),
        pltpu.SemaphoreType.DMA((RB,)),
    ],
  )


def _bn(h, g, b):
    mean = jnp.mean(h, axis=0)
    var = jnp.mean((h - mean) ** 2, axis=0)
    return g * (h - mean) / jnp.sqrt(var + BN_EPS) + b


def _mlp_body(last, p_ref, w1_ref, b1_ref, g1_ref, be1_ref,
              w2_ref, b2_ref, g2_ref, be2_ref, go_ref, bo_ref, out_ref):
    z = p_ref[0, :N, :] + p_ref[1, :N, :]
    y = lax.dot_general(z, w1_ref[...], (((1,), (1,)), ((), ())),
                        preferred_element_type=jnp.float32) + b1_ref[...]
    y = jax.nn.relu(_bn(y, g1_ref[...], be1_ref[...]))
    t = lax.dot_general(y, w2_ref[...], (((1,), (1,)), ((), ())),
                        preferred_element_type=jnp.float32) + b2_ref[...]
    t = _bn(t, g2_ref[...], be2_ref[...])
    h = _bn(t, go_ref[...], bo_ref[...])
    if not last:
        h = jax.nn.relu(h)
    out_ref[:N, :] = h
    out_ref[N:, :] = jnp.zeros((NP - N, D), jnp.float32)


def _tc_mlp(p, w1, b1, g1, be1, w2, b2, g2, be2, go, bo, last):
    return pl.pallas_call(
        functools.partial(_mlp_body, last),
        out_shape=jax.ShapeDtypeStruct((NP, D), jnp.float32),
    )(p, w1, b1, g1, be1, w2, b2, g2, be2, go, bo)


def kernel(x, edge_index, W1, b1, g1, be1, W2, b2, g2, be2, go, bo):
    idx = edge_index.astype(jnp.int32).reshape(2, NW, CH, K).transpose(1, 2, 0, 3)
    z0 = jnp.zeros((RPT, D), jnp.float32)
    h = jnp.pad(x, ((0, NP - N), (0, 0)))
    for i in range(L):
        p = _sc_aggregate()(h, z0, idx)
        h = _tc_mlp(p, W1[i], b1[i], g1[i], be1[i], W2[i], b2[i],
                    g2[i], be2[i], go[i], bo[i], last=(i == L - 1))
    return h[:N]


# speedup vs baseline: 11.0623x; 1.0320x over previous
import functools

import jax
import jax.numpy as jnp
from jax import lax
from jax.experimental import pallas as pl
from jax.experimental.pallas import tpu as pltpu
from jax.experimental.pallas import tpu_sc as plsc

N = 10000
E = 320000
D = 128
L = 3
BN_EPS = 1e-5

NC = 2
NS = 16
NW = NC * NS
EPT = E // NW
K = 40
CH = EPT // K
NBUF = 5
NP = 10240
RPT = NP // NS


RB = 5
IBN = 10
ZR = 64


def _sc_agg_body(h_hbm, src_hbm, dst_hbm, out_hbm,
                 ibuf, rows_v, zbuf, agg_sh, isem, gsem, ssem, zsem):
    cid = lax.axis_index("c")
    sid = lax.axis_index("s")
    wid = cid * NS + sid
    rs = sid * RPT

    def _idx_start(c, s10):
        pltpu.async_copy(src_hbm.at[wid, c], ibuf.at[s10, 0], isem.at[s10])
        pltpu.async_copy(dst_hbm.at[wid, c], ibuf.at[s10, 1], isem.at[s10])

    def _idx_wait(s10):
        pltpu.make_async_copy(src_hbm.at[0, 0], ibuf.at[s10, 0],
                              isem.at[s10]).wait()
        pltpu.make_async_copy(src_hbm.at[0, 0], ibuf.at[s10, 1],
                              isem.at[s10]).wait()

    def _gather_start(s5, s10):
        pltpu.async_copy(h_hbm.at[ibuf.at[s10, 0]], rows_v.at[s5],
                         gsem.at[s5])

    def _gather_wait(s5):
        pltpu.make_async_copy(h_hbm.at[pl.ds(0, K)], rows_v.at[s5],
                              gsem.at[s5]).wait()

    def _scatter_start(s5, s10):
        pltpu.async_copy(rows_v.at[s5], agg_sh.at[ibuf.at[s10, 1]],
                         ssem.at[s5], add=True)

    def _scatter_wait(s5):
        pltpu.make_async_copy(rows_v.at[s5], agg_sh.at[pl.ds(0, K)],
                              ssem.at[s5]).wait()

    zv = jnp.zeros((16,), jnp.float32)
    for r in range(ZR):
        for c16 in range(D // 16):
            zbuf[r, pl.ds(c16 * 16, 16)] = zv

    for j in range(8):
        _idx_start(j, j)
    for j in range(3):
        _idx_wait(j)
        _gather_start(j, j)

    for q in range(RPT // ZR):
        pltpu.async_copy(zbuf, agg_sh.at[pl.ds(rs + q * ZR, ZR)], zsem)
    for q in range(RPT // ZR):
        pltpu.make_async_copy(zbuf, agg_sh.at[pl.ds(rs, ZR)], zsem).wait()
    plsc.subcore_barrier()

    for c in range(2):
        _gather_wait(c % RB)
        _scatter_start(c % RB, c % IBN)
        _idx_start(c + 8, (c + 8) % IBN)
        _idx_wait((c + 3) % IBN)
        _gather_start((c + 3) % RB, (c + 3) % IBN)

    @pl.loop(2, CH - 8, step=IBN)
    def _(cc):
        for b in range(IBN):
            c = cc + b
            s5, s10 = (2 + b) % RB, (2 + b) % IBN
            _gather_wait(s5)
            _scatter_start(s5, s10)
            _scatter_wait((s5 + 3) % RB)
            _idx_start(c + 8, (s10 + 8) % IBN)
            _idx_wait((s10 + 3) % IBN)
            _gather_start((s5 + 3) % RB, (s10 + 3) % IBN)

    for c in range(CH - 8, CH):
        _gather_wait(c % RB)
        _scatter_start(c % RB, c % IBN)
        _scatter_wait((c + 3) % RB)
        if c + 3 < CH:
            _idx_wait((c + 3) % IBN)
            _gather_start((c + 3) % RB, (c + 3) % IBN)
    for c in range(CH - 2, CH):
        _scatter_wait(c % RB)

    plsc.subcore_barrier()
    pltpu.sync_copy(agg_sh.at[pl.ds(rs, RPT)],
                    out_hbm.at[cid, pl.ds(rs, RPT)])


@functools.lru_cache(maxsize=None)
def _sc_aggregate():
  return pl.kernel(
    _sc_agg_body,
    out_type=jax.ShapeDtypeStruct((NC, NP, D), jnp.float32),
    mesh=plsc.VectorSubcoreMesh(core_axis_name="c", subcore_axis_name="s",
                                num_cores=NC, num_subcores=NS),
    scratch_types=[
        pltpu.VMEM((IBN, 2, K), jnp.int32),
        pltpu.VMEM((RB, K, D), jnp.float32),
        pltpu.VMEM((ZR, D), jnp.float32),
        pltpu.VMEM_SHARED((NP, D), jnp.float32),
        pltpu.SemaphoreType.DMA((IBN,)),
        pltpu.SemaphoreType.DMA((RB,)),
        pltpu.SemaphoreType.DMA((RB,)),
        pltpu.SemaphoreType.DMA,
    ],
  )


def _bn(h, g, b):
    mean = jnp.mean(h, axis=0)
    var = jnp.mean((h - mean) ** 2, axis=0)
    return g * (h - mean) / jnp.sqrt(var + BN_EPS) + b


def _mlp_body(last, h_ref, p_ref, w1_ref, b1_ref, g1_ref, be1_ref,
              w2_ref, b2_ref, g2_ref, be2_ref, go_ref, bo_ref, out_ref):
    z = h_ref[:N, :] + p_ref[0, :N, :] + p_ref[1, :N, :]
    y = lax.dot_general(z, w1_ref[...], (((1,), (1,)), ((), ())),
                        preferred_element_type=jnp.float32) + b1_ref[...]
    y = jax.nn.relu(_bn(y, g1_ref[...], be1_ref[...]))
    t = lax.dot_general(y, w2_ref[...], (((1,), (1,)), ((), ())),
                        preferred_element_type=jnp.float32) + b2_ref[...]
    t = _bn(t, g2_ref[...], be2_ref[...])
    h = _bn(t, go_ref[...], bo_ref[...])
    if last:
        out_ref[...] = h
    else:
        out_ref[:N, :] = jax.nn.relu(h)
        out_ref[N:, :] = jnp.zeros((NP - N, D), jnp.float32)


def _tc_mlp(h, p, w1, b1, g1, be1, w2, b2, g2, be2, go, bo, last):
    return pl.pallas_call(
        functools.partial(_mlp_body, last),
        out_shape=jax.ShapeDtypeStruct((N if last else NP, D), jnp.float32),
    )(h, p, w1, b1, g1, be1, w2, b2, g2, be2, go, bo)


def kernel(x, edge_index, W1, b1, g1, be1, W2, b2, g2, be2, go, bo):
    srci = edge_index[0].astype(jnp.int32).reshape(NW, CH, K)
    dsti = edge_index[1].astype(jnp.int32).reshape(NW, CH, K)
    h = jnp.pad(x, ((0, NP - N), (0, 0)))
    for i in range(L):
        p = _sc_aggregate()(h, srci, dsti)
        h = _tc_mlp(h, p, W1[i], b1[i], g1[i], be1[i], W2[i], b2[i],
                    g2[i], be2[i], go[i], bo[i], last=(i == L - 1))
    return h


# speedup vs baseline: 11.7907x; 1.0658x over previous
import functools

import jax
import jax.numpy as jnp
from jax import lax
from jax.experimental import pallas as pl
from jax.experimental.pallas import tpu as pltpu
from jax.experimental.pallas import tpu_sc as plsc

N = 10000
E = 320000
D = 128
L = 3
BN_EPS = 1e-5

NC = 2
NS = 16
NW = NC * NS
EPT = E // NW
K = 40
CH = EPT // K
NBUF = 5
NP = 10240
RPT = NP // NS


RB = 5
IBN = 10
ZR = 64


def _sc_agg_body(h_hbm, ei_hbm, out_hbm,
                 ibuf, rows_v, zbuf, agg_sh, isem, gsem, ssem, zsem):
    cid = lax.axis_index("c")
    sid = lax.axis_index("s")
    wid = cid * NS + sid
    rs = sid * RPT

    def _idx_start(c, s10):
        base = pl.multiple_of(wid * EPT + c * K, 8)
        pltpu.async_copy(ei_hbm.at[pl.ds(base, K)], ibuf.at[s10, 0],
                         isem.at[s10])
        pltpu.async_copy(ei_hbm.at[pl.ds(E + base, K)], ibuf.at[s10, 1],
                         isem.at[s10])

    def _idx_wait(s10):
        pltpu.make_async_copy(ei_hbm.at[pl.ds(0, K)], ibuf.at[s10, 0],
                              isem.at[s10]).wait()
        pltpu.make_async_copy(ei_hbm.at[pl.ds(0, K)], ibuf.at[s10, 1],
                              isem.at[s10]).wait()

    def _gather_start(s5, s10):
        pltpu.async_copy(h_hbm.at[ibuf.at[s10, 0]], rows_v.at[s5],
                         gsem.at[s5])

    def _gather_wait(s5):
        pltpu.make_async_copy(h_hbm.at[pl.ds(0, K)], rows_v.at[s5],
                              gsem.at[s5]).wait()

    def _scatter_start(s5, s10):
        pltpu.async_copy(rows_v.at[s5], agg_sh.at[ibuf.at[s10, 1]],
                         ssem.at[s5], add=True)

    def _scatter_wait(s5):
        pltpu.make_async_copy(rows_v.at[s5], agg_sh.at[pl.ds(0, K)],
                              ssem.at[s5]).wait()

    zv = jnp.zeros((16,), jnp.float32)
    for r in range(ZR):
        for c16 in range(D // 16):
            zbuf[r, pl.ds(c16 * 16, 16)] = zv

    for j in range(8):
        _idx_start(j, j)
    for j in range(3):
        _idx_wait(j)
        _gather_start(j, j)

    for q in range(RPT // ZR):
        pltpu.async_copy(zbuf, agg_sh.at[pl.ds(rs + q * ZR, ZR)], zsem)
    for q in range(RPT // ZR):
        pltpu.make_async_copy(zbuf, agg_sh.at[pl.ds(rs, ZR)], zsem).wait()
    plsc.subcore_barrier()

    for c in range(2):
        _gather_wait(c % RB)
        _scatter_start(c % RB, c % IBN)
        _idx_start(c + 8, (c + 8) % IBN)
        _idx_wait((c + 3) % IBN)
        _gather_start((c + 3) % RB, (c + 3) % IBN)

    @pl.loop(2, CH - 8, step=IBN)
    def _(cc):
        for b in range(IBN):
            c = cc + b
            s5, s10 = (2 + b) % RB, (2 + b) % IBN
            _gather_wait(s5)
            _scatter_start(s5, s10)
            _scatter_wait((s5 + 3) % RB)
            _idx_start(c + 8, (s10 + 8) % IBN)
            _idx_wait((s10 + 3) % IBN)
            _gather_start((s5 + 3) % RB, (s10 + 3) % IBN)

    for c in range(CH - 8, CH):
        _gather_wait(c % RB)
        _scatter_start(c % RB, c % IBN)
        _scatter_wait((c + 3) % RB)
        if c + 3 < CH:
            _idx_wait((c + 3) % IBN)
            _gather_start((c + 3) % RB, (c + 3) % IBN)
    for c in range(CH - 2, CH):
        _scatter_wait(c % RB)

    plsc.subcore_barrier()
    pltpu.sync_copy(agg_sh.at[pl.ds(rs, RPT)],
                    out_hbm.at[cid, pl.ds(rs, RPT)])


@functools.lru_cache(maxsize=None)
def _sc_aggregate():
  return pl.kernel(
    _sc_agg_body,
    out_type=jax.ShapeDtypeStruct((NC, NP, D), jnp.float32),
    mesh=plsc.VectorSubcoreMesh(core_axis_name="c", subcore_axis_name="s",
                                num_cores=NC, num_subcores=NS),
    scratch_types=[
        pltpu.VMEM((IBN, 2, K), jnp.int32),
        pltpu.VMEM((RB, K, D), jnp.float32),
        pltpu.VMEM((ZR, D), jnp.float32),
        pltpu.VMEM_SHARED((NP, D), jnp.float32),
        pltpu.SemaphoreType.DMA((IBN,)),
        pltpu.SemaphoreType.DMA((RB,)),
        pltpu.SemaphoreType.DMA((RB,)),
        pltpu.SemaphoreType.DMA,
    ],
  )


def _bn(h, g, b):
    mean = jnp.mean(h, axis=0)
    var = jnp.mean((h - mean) ** 2, axis=0)
    return g * (h - mean) / jnp.sqrt(var + BN_EPS) + b


def _mlp_body(last, h_ref, p_ref, w1_ref, b1_ref, g1_ref, be1_ref,
              w2_ref, b2_ref, g2_ref, be2_ref, go_ref, bo_ref, out_ref):
    z = h_ref[...] + p_ref[0, :N, :] + p_ref[1, :N, :]
    y = lax.dot_general(z, w1_ref[...], (((1,), (1,)), ((), ())),
                        preferred_element_type=jnp.float32) + b1_ref[...]
    y = jax.nn.relu(_bn(y, g1_ref[...], be1_ref[...]))
    t = lax.dot_general(y, w2_ref[...], (((1,), (1,)), ((), ())),
                        preferred_element_type=jnp.float32) + b2_ref[...]
    t = _bn(t, g2_ref[...], be2_ref[...])
    h = _bn(t, go_ref[...], bo_ref[...])
    if not last:
        h = jax.nn.relu(h)
    out_ref[...] = h


def _tc_mlp(h, p, w1, b1, g1, be1, w2, b2, g2, be2, go, bo, last):
    return pl.pallas_call(
        functools.partial(_mlp_body, last),
        out_shape=jax.ShapeDtypeStruct((N, D), jnp.float32),
    )(h, p, w1, b1, g1, be1, w2, b2, g2, be2, go, bo)


def kernel(x, edge_index, W1, b1, g1, be1, W2, b2, g2, be2, go, bo):
    ei = edge_index.astype(jnp.int32).reshape(2 * E)
    h = x
    for i in range(L):
        p = _sc_aggregate()(h, ei)
        h = _tc_mlp(h, p, W1[i], b1[i], g1[i], be1[i], W2[i], b2[i],
                    g2[i], be2[i], go[i], bo[i], last=(i == L - 1))
    return h
